# Initial kernel scaffold; baseline (speedup 1.0000x reference)
import jax
import jax.numpy as jnp
from jax.experimental import pallas as pl


def kernel(queries, keys, values, attn_mask, tau, delta, W_qk, W_v, W_out, b_out):
    raise NotImplementedError("write your pallas kernel here")


# speedup vs baseline: 4.3106x; 4.3106x over previous
import functools

import jax
import jax.numpy as jnp
from jax import lax
from jax.experimental import pallas as pl
from jax.experimental.pallas import tpu as pltpu
from jax.experimental.pallas import tpu_sc as plsc

N = 8192
D = 768
H = 12
DH = 64
BUCKET = 64
NH = 4
NB = N // BUCKET
NCH = NH * NB
TOT = NH * N

_RB = 512
_CPB = 8
_SO_D = 128


def _proj_hash_body(q_ref, wqk_ref, wv_ref, rot_ref, qkv_ref, keys_ref):
    q = q_ref[...]
    qk = jnp.dot(q, wqk_ref[...], preferred_element_type=jnp.float32)
    v = jnp.dot(q, wv_ref[...], preferred_element_type=jnp.float32)
    row0 = pl.program_id(0) * _RB
    pos = row0 + lax.broadcasted_iota(jnp.int32, (_RB, 1), 0)
    lane = lax.broadcasted_iota(jnp.int32, (_RB, 128), 1)
    for h in range(H):
        qk_h = qk[:, DH * h:DH * (h + 1)]
        qkv_ref[:, 128 * h:128 * h + 64] = qk_h
        qkv_ref[:, 128 * h + 64:128 * h + 128] = v[:, DH * h:DH * (h + 1)]
        r = jnp.dot(qk_h, rot_ref[...], preferred_element_type=jnp.float32)
        for nh in range(NH):
            g = r[:, 128 * nh:128 * nh + 128]
            m = jnp.max(g, axis=1, keepdims=True)
            idx = jnp.min(jnp.where(g >= m, lane, 128), axis=1, keepdims=True)
            key = (idx + 128 * nh) * N + pos
            c = NH * h + nh
            keys_ref[:, c:c + 1] = key
    del v


def _proj_hash(queries2d, W_qk, W_v, rot_cat):
    grid = (N // _RB,)
    return pl.pallas_call(
        _proj_hash_body,
        grid=grid,
        in_specs=[
            pl.BlockSpec((_RB, D), lambda i: (i, 0)),
            pl.BlockSpec((D, D), lambda i: (0, 0)),
            pl.BlockSpec((D, D), lambda i: (0, 0)),
            pl.BlockSpec((DH, NH * 128), lambda i: (0, 0)),
        ],
        out_specs=[
            pl.BlockSpec((_RB, H * 128), lambda i: (i, 0)),
            pl.BlockSpec((_RB, H * NH), lambda i: (i, 0)),
        ],
        out_shape=[
            jax.ShapeDtypeStruct((N, H * 128), jnp.float32),
            jax.ShapeDtypeStruct((N, H * NH), jnp.int32),
        ],
    )(queries2d, W_qk, W_v, rot_cat)


def _sc_gather(table, idx, d_row, chunk):
    btot = idx.shape[0]
    nw = 32
    bpw = btot // nw
    steps = bpw // chunk
    mesh = plsc.VectorSubcoreMesh(core_axis_name="c", subcore_axis_name="s")

    @functools.partial(
        pl.kernel, mesh=mesh,
        out_type=jax.ShapeDtypeStruct((btot, d_row), jnp.float32),
        scratch_types=[
            pltpu.VMEM((chunk,), jnp.int32),
            pltpu.VMEM((chunk, d_row), jnp.float32),
            pltpu.SemaphoreType.DMA,
        ],
    )
    def k(table_hbm, idx_hbm, out_hbm, idx_v, rows_v, sem):
        wid = lax.axis_index("s") * 2 + lax.axis_index("c")
        base = wid * bpw

        def body(i, carry):
            off = base + i * chunk
            pltpu.sync_copy(idx_hbm.at[pl.ds(off, chunk)], idx_v)
            pltpu.async_copy(table_hbm.at[idx_v], rows_v, sem).wait()
            pltpu.sync_copy(rows_v, out_hbm.at[pl.ds(off, chunk)])
            return carry

        lax.fori_loop(0, steps, body, 0)

    return k(table, idx)


def _attn_body(sqkv_c, sqkv_p, strow_c, strow_p, stcol_c, so_ref):
    cur = sqkv_c[0]
    prevb = sqkv_p[0]
    trow_c = strow_c[0, 0]
    trow_p = strow_p[0, 0]
    tcol = stcol_c[0, 0]
    scale = DH ** -0.5
    for j in range(_CPB):
        cur_chunk = cur[64 * j:64 * (j + 1), :]
        prev_chunk = prevb if j == 0 else cur[64 * (j - 1):64 * j, :]
        kv = jnp.concatenate([cur_chunk, prev_chunk], axis=0)
        bq = cur_chunk[:, 0:64]
        bk = kv[:, 0:64]
        bv = kv[:, 64:128]
        nrm = jnp.sqrt(jnp.sum(bk * bk, axis=1, keepdims=True))
        bk = bk / jnp.maximum(nrm, 1e-6)
        dots = lax.dot_general(bq, bk, (((1,), (1,)), ((), ())),
                               preferred_element_type=jnp.float32) * scale
        bq_t = tcol[64 * j:64 * (j + 1), :]
        kt_cur = trow_c[:, 64 * j:64 * (j + 1)]
        kt_prev = trow_p[:, 448:512] if j == 0 else trow_c[:, 64 * (j - 1):64 * j]
        bkv_t = jnp.concatenate([kt_cur, kt_prev], axis=1)
        dots = dots - 1e5 * (bq_t == bkv_t).astype(jnp.float32)
        m = jnp.max(dots, axis=1, keepdims=True)
        p = jnp.exp(dots - m)
        s = jnp.sum(p, axis=1, keepdims=True)
        lse = m + jnp.log(s)
        bo = lax.dot_general(p / s, bv, (((1,), (0,)), ((), ())),
                             preferred_element_type=jnp.float32)
        so_ref[0, 64 * j:64 * (j + 1), 0:64] = bo
        so_ref[0, 64 * j:64 * (j + 1), 64:65] = lse
        so_ref[0, 64 * j:64 * (j + 1), 65:_SO_D] = jnp.zeros((64, _SO_D - 65),
                                                             jnp.float32)


def _attention(sqkv, st_row4, st_col4):
    nblk = NCH // _CPB
    rows = _CPB * BUCKET
    return pl.pallas_call(
        _attn_body,
        grid=(H, nblk),
        in_specs=[
            pl.BlockSpec((1, rows, 128), lambda h, i: (h, i, 0)),
            pl.BlockSpec((1, 64, 128), lambda h, i: (h, (_CPB * i - 1) % NCH, 0)),
            pl.BlockSpec((1, 1, 1, rows), lambda h, i: (h, i, 0, 0)),
            pl.BlockSpec((1, 1, 1, rows), lambda h, i: (h, (i - 1) % nblk, 0, 0)),
            pl.BlockSpec((1, 1, rows, 1), lambda h, i: (h, i, 0, 0)),
        ],
        out_specs=pl.BlockSpec((1, rows, _SO_D), lambda h, i: (h, i, 0)),
        out_shape=jax.ShapeDtypeStruct((H, TOT, _SO_D), jnp.float32),
    )(sqkv, sqkv, st_row4, st_row4, st_col4)


def _combine_body(o_ref, w_ref, b_ref, out_ref):
    h = pl.program_id(1)
    blk = o_ref[0]
    logit = [blk[i, :, 64:65] for i in range(NH)]
    m = jnp.maximum(jnp.maximum(logit[0], logit[1]),
                    jnp.maximum(logit[2], logit[3]))
    e = [jnp.exp(x - m) for x in logit]
    s = e[0] + e[1] + e[2] + e[3]
    comb = sum(blk[i, :, 0:64] * (e[i] / s) for i in range(NH))
    contrib = jnp.dot(comb, w_ref[...], preferred_element_type=jnp.float32)

    @pl.when(h == 0)
    def _():
        out_ref[...] = contrib + b_ref[...]

    @pl.when(h != 0)
    def _():
        out_ref[...] += contrib


def _combine(o_uns4, W_out, b_out2):
    return pl.pallas_call(
        _combine_body,
        grid=(N // _RB, H),
        in_specs=[
            pl.BlockSpec((1, NH, _RB, _SO_D), lambda i, h: (h, 0, i, 0)),
            pl.BlockSpec((DH, D), lambda i, h: (h, 0)),
            pl.BlockSpec((1, D), lambda i, h: (0, 0)),
        ],
        out_specs=pl.BlockSpec((_RB, D), lambda i, h: (i, 0)),
        out_shape=jax.ShapeDtypeStruct((N, D), jnp.float32),
    )(o_uns4, W_out, b_out2)


def kernel(queries, keys, values, attn_mask, tau, delta, W_qk, W_v, W_out, b_out):
    del keys, values, attn_mask, tau, delta
    q2 = queries.reshape(N, D)

    rot = jax.random.normal(jax.random.key(42), (DH, NH, NB // 2), jnp.float32)
    rot_cat = jnp.concatenate([rot, -rot], axis=-1).reshape(DH, NH * 128)

    qkv, keys_tok = _proj_hash(q2, W_qk, W_v, rot_cat)

    keys_hm = keys_tok.reshape(N, H, NH).transpose(1, 2, 0).reshape(H, TOT)
    sticker = jnp.argsort(keys_hm, axis=-1).astype(jnp.int32)
    undo = jnp.argsort(sticker, axis=-1).astype(jnp.int32)
    st = sticker % N

    gidx1 = (st * H + jnp.arange(H, dtype=jnp.int32)[:, None]).reshape(-1)
    sqkv = _sc_gather(qkv.reshape(N * H, 128), gidx1, 128, 512)
    sqkv = sqkv.reshape(H, TOT, 128)

    st_row4 = st.reshape(H, NCH // _CPB, 1, _CPB * BUCKET)
    st_col4 = st.reshape(H, NCH // _CPB, _CPB * BUCKET, 1)
    so = _attention(sqkv, st_row4, st_col4)

    gidx2 = (undo + (jnp.arange(H, dtype=jnp.int32) * TOT)[:, None]).reshape(-1)
    o_uns = _sc_gather(so.reshape(H * TOT, _SO_D), gidx2, _SO_D, 512)
    o_uns4 = o_uns.reshape(H, NH, N, _SO_D)

    out = _combine(o_uns4, W_out, b_out.reshape(1, D))
    return out.reshape(1, N, D)


# speedup vs baseline: 4.5906x; 1.0650x over previous
import functools

import jax
import jax.numpy as jnp
from jax import lax
from jax.experimental import pallas as pl
from jax.experimental.pallas import tpu as pltpu
from jax.experimental.pallas import tpu_sc as plsc

N = 8192
D = 768
H = 12
DH = 64
BUCKET = 64
NH = 4
NB = N // BUCKET
NCH = NH * NB
TOT = NH * N

_RB = 512
_CPB = 8
_SO_D = 128


def _proj_hash_body(q_ref, wqk_ref, wv_ref, rot_ref, qkv_ref, keys_ref):
    q = q_ref[...]
    qk = jnp.dot(q, wqk_ref[...], preferred_element_type=jnp.float32)
    v = jnp.dot(q, wv_ref[...], preferred_element_type=jnp.float32)
    row0 = pl.program_id(0) * _RB
    pos = row0 + lax.broadcasted_iota(jnp.int32, (_RB, 1), 0)
    lane = lax.broadcasted_iota(jnp.int32, (_RB, 128), 1)
    for h in range(H):
        qk_h = qk[:, DH * h:DH * (h + 1)]
        qkv_ref[:, 128 * h:128 * h + 64] = qk_h
        qkv_ref[:, 128 * h + 64:128 * h + 128] = v[:, DH * h:DH * (h + 1)]
        r = jnp.dot(qk_h, rot_ref[...], preferred_element_type=jnp.float32)
        for nh in range(NH):
            g = r[:, 128 * nh:128 * nh + 128]
            m = jnp.max(g, axis=1, keepdims=True)
            idx = jnp.min(jnp.where(g >= m, lane, 128), axis=1, keepdims=True)
            key = (idx + 128 * nh) * N + pos
            c = NH * h + nh
            keys_ref[:, c:c + 1] = key
    del v


def _proj_hash(queries2d, W_qk, W_v, rot_cat):
    grid = (N // _RB,)
    return pl.pallas_call(
        _proj_hash_body,
        grid=grid,
        in_specs=[
            pl.BlockSpec((_RB, D), lambda i: (i, 0)),
            pl.BlockSpec((D, D), lambda i: (0, 0)),
            pl.BlockSpec((D, D), lambda i: (0, 0)),
            pl.BlockSpec((DH, NH * 128), lambda i: (0, 0)),
        ],
        out_specs=[
            pl.BlockSpec((_RB, H * 128), lambda i: (i, 0)),
            pl.BlockSpec((_RB, H * NH), lambda i: (i, 0)),
        ],
        out_shape=[
            jax.ShapeDtypeStruct((N, H * 128), jnp.float32),
            jax.ShapeDtypeStruct((N, H * NH), jnp.int32),
        ],
    )(queries2d, W_qk, W_v, rot_cat)


def _sc_gather(table, idx, d_row, chunk):
    btot = idx.shape[0]
    nw = 32
    bpw = btot // nw
    steps = bpw // chunk
    mesh = plsc.VectorSubcoreMesh(core_axis_name="c", subcore_axis_name="s")

    @functools.partial(
        pl.kernel, mesh=mesh,
        out_type=jax.ShapeDtypeStruct((btot, d_row), jnp.float32),
        scratch_types=[
            pltpu.VMEM((chunk,), jnp.int32),
            pltpu.VMEM((chunk, d_row), jnp.float32),
            pltpu.SemaphoreType.DMA,
        ],
    )
    def k(table_hbm, idx_hbm, out_hbm, idx_v, rows_v, sem):
        wid = lax.axis_index("s") * 2 + lax.axis_index("c")
        base = wid * bpw

        def body(i, carry):
            off = base + i * chunk
            pltpu.sync_copy(idx_hbm.at[pl.ds(off, chunk)], idx_v)
            pltpu.async_copy(table_hbm.at[idx_v], rows_v, sem).wait()
            pltpu.sync_copy(rows_v, out_hbm.at[pl.ds(off, chunk)])
            return carry

        lax.fori_loop(0, steps, body, 0)

    return k(table, idx)


def _sc_scatter(src, idx, d_row, chunk):
    btot = idx.shape[0]
    nw = 32
    bpw = btot // nw
    steps = bpw // chunk
    mesh = plsc.VectorSubcoreMesh(core_axis_name="c", subcore_axis_name="s")

    @functools.partial(
        pl.kernel, mesh=mesh,
        out_type=jax.ShapeDtypeStruct((btot, d_row), jnp.float32),
        scratch_types=[
            pltpu.VMEM((chunk,), jnp.int32),
            pltpu.VMEM((chunk, d_row), jnp.float32),
            pltpu.SemaphoreType.DMA,
        ],
    )
    def k(src_hbm, idx_hbm, out_hbm, idx_v, rows_v, sem):
        wid = lax.axis_index("s") * 2 + lax.axis_index("c")
        base = wid * bpw

        def body(i, carry):
            off = base + i * chunk
            pltpu.sync_copy(idx_hbm.at[pl.ds(off, chunk)], idx_v)
            pltpu.sync_copy(src_hbm.at[pl.ds(off, chunk)], rows_v)
            pltpu.async_copy(rows_v, out_hbm.at[idx_v], sem).wait()
            return carry

        lax.fori_loop(0, steps, body, 0)

    return k(src, idx)


def _attn_body(sqkv_c, sqkv_p, strow_c, strow_p, stcol_c, so_ref):
    cur = sqkv_c[0]
    prevb = sqkv_p[0]
    trow_c = strow_c[0, 0]
    trow_p = strow_p[0, 0]
    tcol = stcol_c[0, 0]
    scale = DH ** -0.5
    for j in range(_CPB):
        cur_chunk = cur[64 * j:64 * (j + 1), :]
        prev_chunk = prevb if j == 0 else cur[64 * (j - 1):64 * j, :]
        kv = jnp.concatenate([cur_chunk, prev_chunk], axis=0)
        bq = cur_chunk[:, 0:64]
        bk = kv[:, 0:64]
        bv = kv[:, 64:128]
        nrm = jnp.sqrt(jnp.sum(bk * bk, axis=1, keepdims=True))
        bk = bk / jnp.maximum(nrm, 1e-6)
        dots = lax.dot_general(bq, bk, (((1,), (1,)), ((), ())),
                               preferred_element_type=jnp.float32) * scale
        bq_t = tcol[64 * j:64 * (j + 1), :]
        kt_cur = trow_c[:, 64 * j:64 * (j + 1)]
        kt_prev = trow_p[:, 448:512] if j == 0 else trow_c[:, 64 * (j - 1):64 * j]
        bkv_t = jnp.concatenate([kt_cur, kt_prev], axis=1)
        dots = dots - 1e5 * (bq_t == bkv_t).astype(jnp.float32)
        m = jnp.max(dots, axis=1, keepdims=True)
        p = jnp.exp(dots - m)
        s = jnp.sum(p, axis=1, keepdims=True)
        lse = m + jnp.log(s)
        bo = lax.dot_general(p / s, bv, (((1,), (0,)), ((), ())),
                             preferred_element_type=jnp.float32)
        so_ref[0, 64 * j:64 * (j + 1), 0:64] = bo
        so_ref[0, 64 * j:64 * (j + 1), 64:65] = lse
        so_ref[0, 64 * j:64 * (j + 1), 65:_SO_D] = jnp.zeros((64, _SO_D - 65),
                                                             jnp.float32)


def _attention(sqkv, st_row4, st_col4):
    nblk = NCH // _CPB
    rows = _CPB * BUCKET
    return pl.pallas_call(
        _attn_body,
        grid=(H, nblk),
        in_specs=[
            pl.BlockSpec((1, rows, 128), lambda h, i: (h, i, 0)),
            pl.BlockSpec((1, 64, 128), lambda h, i: (h, (_CPB * i - 1) % NCH, 0)),
            pl.BlockSpec((1, 1, 1, rows), lambda h, i: (h, i, 0, 0)),
            pl.BlockSpec((1, 1, 1, rows), lambda h, i: (h, (i - 1) % nblk, 0, 0)),
            pl.BlockSpec((1, 1, rows, 1), lambda h, i: (h, i, 0, 0)),
        ],
        out_specs=pl.BlockSpec((1, rows, _SO_D), lambda h, i: (h, i, 0)),
        out_shape=jax.ShapeDtypeStruct((H, TOT, _SO_D), jnp.float32),
    )(sqkv, sqkv, st_row4, st_row4, st_col4)


def _combine_body(o_ref, w_ref, b_ref, out_ref):
    h = pl.program_id(1)
    blk = o_ref[0]
    logit = [blk[i, :, 64:65] for i in range(NH)]
    m = jnp.maximum(jnp.maximum(logit[0], logit[1]),
                    jnp.maximum(logit[2], logit[3]))
    e = [jnp.exp(x - m) for x in logit]
    s = e[0] + e[1] + e[2] + e[3]
    comb = sum(blk[i, :, 0:64] * (e[i] / s) for i in range(NH))
    contrib = jnp.dot(comb, w_ref[...], preferred_element_type=jnp.float32)

    @pl.when(h == 0)
    def _():
        out_ref[...] = contrib + b_ref[...]

    @pl.when(h != 0)
    def _():
        out_ref[...] += contrib


def _combine(o_uns4, W_out, b_out2):
    return pl.pallas_call(
        _combine_body,
        grid=(N // _RB, H),
        in_specs=[
            pl.BlockSpec((1, NH, _RB, _SO_D), lambda i, h: (h, 0, i, 0)),
            pl.BlockSpec((DH, D), lambda i, h: (h, 0)),
            pl.BlockSpec((1, D), lambda i, h: (0, 0)),
        ],
        out_specs=pl.BlockSpec((_RB, D), lambda i, h: (i, 0)),
        out_shape=jax.ShapeDtypeStruct((N, D), jnp.float32),
    )(o_uns4, W_out, b_out2)


def kernel(queries, keys, values, attn_mask, tau, delta, W_qk, W_v, W_out, b_out):
    del keys, values, attn_mask, tau, delta
    q2 = queries.reshape(N, D)

    rot = jax.random.normal(jax.random.key(42), (DH, NH, NB // 2), jnp.float32)
    rot_cat = jnp.concatenate([rot, -rot], axis=-1).reshape(DH, NH * 128)

    qkv, keys_tok = _proj_hash(q2, W_qk, W_v, rot_cat)

    keys_hm = keys_tok.reshape(N, H, NH).transpose(1, 2, 0).reshape(H, TOT)
    sticker = jnp.argsort(keys_hm, axis=-1).astype(jnp.int32)
    st = sticker % N

    gidx1 = (st * H + jnp.arange(H, dtype=jnp.int32)[:, None]).reshape(-1)
    sqkv = _sc_gather(qkv.reshape(N * H, 128), gidx1, 128, 512)
    sqkv = sqkv.reshape(H, TOT, 128)

    st_row4 = st.reshape(H, NCH // _CPB, 1, _CPB * BUCKET)
    st_col4 = st.reshape(H, NCH // _CPB, _CPB * BUCKET, 1)
    so = _attention(sqkv, st_row4, st_col4)

    scidx = (sticker + (jnp.arange(H, dtype=jnp.int32) * TOT)[:, None]).reshape(-1)
    o_uns = _sc_scatter(so.reshape(H * TOT, _SO_D), scidx, _SO_D, 512)
    o_uns4 = o_uns.reshape(H, NH, N, _SO_D)

    out = _combine(o_uns4, W_out, b_out.reshape(1, D))
    return out.reshape(1, N, D)


# speedup vs baseline: 6.3038x; 1.3732x over previous
import functools

import jax
import jax.numpy as jnp
from jax import lax
from jax.experimental import pallas as pl
from jax.experimental.pallas import tpu as pltpu
from jax.experimental.pallas import tpu_sc as plsc

N = 8192
D = 768
H = 12
DH = 64
BUCKET = 64
NH = 4
NB = N // BUCKET
NCH = NH * NB
TOT = NH * N

_RB = 512
_CPB = 8
_SO_D = 128


def _proj_hash_body(q_ref, wqk_ref, wv_ref, rot_ref, qkv_ref, keys_ref):
    q = q_ref[...]
    qk = jnp.dot(q, wqk_ref[...], preferred_element_type=jnp.float32)
    v = jnp.dot(q, wv_ref[...], preferred_element_type=jnp.float32)
    row0 = pl.program_id(0) * _RB
    pos = row0 + lax.broadcasted_iota(jnp.int32, (_RB, 1), 0)
    lane = lax.broadcasted_iota(jnp.int32, (_RB, 128), 1)
    for h in range(H):
        qk_h = qk[:, DH * h:DH * (h + 1)]
        qkv_ref[:, 128 * h:128 * h + 64] = qk_h
        qkv_ref[:, 128 * h + 64:128 * h + 128] = v[:, DH * h:DH * (h + 1)]
        r = jnp.dot(qk_h, rot_ref[...], preferred_element_type=jnp.float32)
        for nh in range(NH):
            g = r[:, 128 * nh:128 * nh + 128]
            m = jnp.max(g, axis=1, keepdims=True)
            idx = jnp.min(jnp.where(g >= m, lane, 128), axis=1, keepdims=True)
            key = (idx + 128 * nh) * N + pos
            c = NH * h + nh
            keys_ref[:, c:c + 1] = key
    del v


def _proj_hash(queries2d, W_qk, W_v, rot_cat):
    grid = (N // _RB,)
    return pl.pallas_call(
        _proj_hash_body,
        grid=grid,
        in_specs=[
            pl.BlockSpec((_RB, D), lambda i: (i, 0)),
            pl.BlockSpec((D, D), lambda i: (0, 0)),
            pl.BlockSpec((D, D), lambda i: (0, 0)),
            pl.BlockSpec((DH, NH * 128), lambda i: (0, 0)),
        ],
        out_specs=[
            pl.BlockSpec((_RB, H * 128), lambda i: (i, 0)),
            pl.BlockSpec((_RB, H * NH), lambda i: (i, 0)),
        ],
        out_shape=[
            jax.ShapeDtypeStruct((N, H * 128), jnp.float32),
            jax.ShapeDtypeStruct((N, H * NH), jnp.int32),
        ],
    )(queries2d, W_qk, W_v, rot_cat)


def _sc_gather(table, idx, d_row, chunk):
    btot = idx.shape[0]
    nw = 32
    bpw = btot // nw
    steps = bpw // chunk
    mesh = plsc.VectorSubcoreMesh(core_axis_name="c", subcore_axis_name="s")

    @functools.partial(
        pl.kernel, mesh=mesh,
        out_type=jax.ShapeDtypeStruct((btot, d_row), jnp.float32),
        scratch_types=[
            pltpu.VMEM((chunk,), jnp.int32),
            pltpu.VMEM((chunk, d_row), jnp.float32),
            pltpu.SemaphoreType.DMA,
        ],
    )
    def k(table_hbm, idx_hbm, out_hbm, idx_v, rows_v, sem):
        wid = lax.axis_index("s") * 2 + lax.axis_index("c")
        base = wid * bpw

        def body(i, carry):
            off = base + i * chunk
            pltpu.sync_copy(idx_hbm.at[pl.ds(off, chunk)], idx_v)
            pltpu.async_copy(table_hbm.at[idx_v], rows_v, sem).wait()
            pltpu.sync_copy(rows_v, out_hbm.at[pl.ds(off, chunk)])
            return carry

        lax.fori_loop(0, steps, body, 0)

    return k(table, idx)


def _sc_scatter(src, idx, d_row, chunk):
    btot = idx.shape[0]
    nw = 32
    bpw = btot // nw
    steps = bpw // chunk
    mesh = plsc.VectorSubcoreMesh(core_axis_name="c", subcore_axis_name="s")

    @functools.partial(
        pl.kernel, mesh=mesh,
        out_type=jax.ShapeDtypeStruct((btot, d_row), jnp.float32),
        scratch_types=[
            pltpu.VMEM((chunk,), jnp.int32),
            pltpu.VMEM((chunk, d_row), jnp.float32),
            pltpu.SemaphoreType.DMA,
        ],
    )
    def k(src_hbm, idx_hbm, out_hbm, idx_v, rows_v, sem):
        wid = lax.axis_index("s") * 2 + lax.axis_index("c")
        base = wid * bpw

        def body(i, carry):
            off = base + i * chunk
            pltpu.sync_copy(idx_hbm.at[pl.ds(off, chunk)], idx_v)
            pltpu.sync_copy(src_hbm.at[pl.ds(off, chunk)], rows_v)
            pltpu.async_copy(rows_v, out_hbm.at[idx_v], sem).wait()
            return carry

        lax.fori_loop(0, steps, body, 0)

    return k(src, idx)


def _attn_body(sqkv_c, sqkv_p, strow_c, strow_p, stcol_c, so_ref):
    rows = _CPB * BUCKET
    ext_n = rows + BUCKET
    cur = sqkv_c[0]
    prevb = sqkv_p[0]
    trow_c = strow_c[0, 0]
    trow_p = strow_p[0, 0]
    tcol = stcol_c[0, 0]
    scale = DH ** -0.5

    ext = jnp.concatenate([prevb, cur], axis=0)
    ext_k = ext[:, 0:64]
    ext_v = ext[:, 64:128]
    nrm = jnp.sqrt(jnp.sum(ext_k * ext_k, axis=1, keepdims=True))
    ext_k = ext_k / jnp.maximum(nrm, 1e-6)
    bq = cur[:, 0:64]

    dots = lax.dot_general(bq, ext_k, (((1,), (1,)), ((), ())),
                           preferred_element_type=jnp.float32) * scale
    ext_t = jnp.concatenate([trow_p[:, rows - 64:rows], trow_c], axis=1)
    dots = dots - 1e5 * (tcol == ext_t).astype(jnp.float32)
    rb = lax.broadcasted_iota(jnp.int32, (rows, ext_n), 0) // BUCKET
    cb = lax.broadcasted_iota(jnp.int32, (rows, ext_n), 1) // BUCKET
    valid = (cb == rb) | (cb == rb + 1)
    dots = jnp.where(valid, dots, -1e9)
    m = jnp.max(dots, axis=1, keepdims=True)
    p = jnp.exp(dots - m)
    s = jnp.sum(p, axis=1, keepdims=True)
    lse = m + jnp.log(s)
    bo = lax.dot_general(p / s, ext_v, (((1,), (0,)), ((), ())),
                         preferred_element_type=jnp.float32)
    so_ref[0, :, 0:64] = bo
    so_ref[0, :, 64:65] = lse
    so_ref[0, :, 65:_SO_D] = jnp.zeros((rows, _SO_D - 65), jnp.float32)


def _attention(sqkv, st_row4, st_col4):
    nblk = NCH // _CPB
    rows = _CPB * BUCKET
    return pl.pallas_call(
        _attn_body,
        grid=(H, nblk),
        in_specs=[
            pl.BlockSpec((1, rows, 128), lambda h, i: (h, i, 0)),
            pl.BlockSpec((1, 64, 128), lambda h, i: (h, (_CPB * i - 1) % NCH, 0)),
            pl.BlockSpec((1, 1, 1, rows), lambda h, i: (h, i, 0, 0)),
            pl.BlockSpec((1, 1, 1, rows), lambda h, i: (h, (i - 1) % nblk, 0, 0)),
            pl.BlockSpec((1, 1, rows, 1), lambda h, i: (h, i, 0, 0)),
        ],
        out_specs=pl.BlockSpec((1, rows, _SO_D), lambda h, i: (h, i, 0)),
        out_shape=jax.ShapeDtypeStruct((H, TOT, _SO_D), jnp.float32),
    )(sqkv, sqkv, st_row4, st_row4, st_col4)


def _combine_body(o_ref, w_ref, b_ref, out_ref):
    h = pl.program_id(1)
    blk = o_ref[0]
    logit = [blk[i, :, 64:65] for i in range(NH)]
    m = jnp.maximum(jnp.maximum(logit[0], logit[1]),
                    jnp.maximum(logit[2], logit[3]))
    e = [jnp.exp(x - m) for x in logit]
    s = e[0] + e[1] + e[2] + e[3]
    comb = sum(blk[i, :, 0:64] * (e[i] / s) for i in range(NH))
    contrib = jnp.dot(comb, w_ref[...], preferred_element_type=jnp.float32)

    @pl.when(h == 0)
    def _():
        out_ref[...] = contrib + b_ref[...]

    @pl.when(h != 0)
    def _():
        out_ref[...] += contrib


def _combine(o_uns4, W_out, b_out2):
    return pl.pallas_call(
        _combine_body,
        grid=(N // _RB, H),
        in_specs=[
            pl.BlockSpec((1, NH, _RB, _SO_D), lambda i, h: (h, 0, i, 0)),
            pl.BlockSpec((DH, D), lambda i, h: (h, 0)),
            pl.BlockSpec((1, D), lambda i, h: (0, 0)),
        ],
        out_specs=pl.BlockSpec((_RB, D), lambda i, h: (i, 0)),
        out_shape=jax.ShapeDtypeStruct((N, D), jnp.float32),
    )(o_uns4, W_out, b_out2)


def kernel(queries, keys, values, attn_mask, tau, delta, W_qk, W_v, W_out, b_out):
    del keys, values, attn_mask, tau, delta
    q2 = queries.reshape(N, D)

    rot = jax.random.normal(jax.random.key(42), (DH, NH, NB // 2), jnp.float32)
    rot_cat = jnp.concatenate([rot, -rot], axis=-1).reshape(DH, NH * 128)

    qkv, keys_tok = _proj_hash(q2, W_qk, W_v, rot_cat)

    keys_hm = keys_tok.reshape(N, H, NH).transpose(1, 2, 0).reshape(H, TOT)
    sticker = jnp.argsort(keys_hm, axis=-1).astype(jnp.int32)
    st = sticker % N

    gidx1 = (st * H + jnp.arange(H, dtype=jnp.int32)[:, None]).reshape(-1)
    sqkv = _sc_gather(qkv.reshape(N * H, 128), gidx1, 128, 512)
    sqkv = sqkv.reshape(H, TOT, 128)

    st_row4 = st.reshape(H, NCH // _CPB, 1, _CPB * BUCKET)
    st_col4 = st.reshape(H, NCH // _CPB, _CPB * BUCKET, 1)
    so = _attention(sqkv, st_row4, st_col4)

    scidx = (sticker + (jnp.arange(H, dtype=jnp.int32) * TOT)[:, None]).reshape(-1)
    o_uns = _sc_scatter(so.reshape(H * TOT, _SO_D), scidx, _SO_D, 512)
    o_uns4 = o_uns.reshape(H, NH, N, _SO_D)

    out = _combine(o_uns4, W_out, b_out.reshape(1, D))
    return out.reshape(1, N, D)


# speedup vs baseline: 7.4705x; 1.1851x over previous
import functools

import jax
import jax.numpy as jnp
from jax import lax
from jax.experimental import pallas as pl
from jax.experimental.pallas import tpu as pltpu
from jax.experimental.pallas import tpu_sc as plsc

N = 8192
D = 768
H = 12
DH = 64
BUCKET = 64
NH = 4
NB = N // BUCKET
NCH = NH * NB
TOT = NH * N

_RB = 512
_CPB = 8
_SO_D = 128


def _proj_hash_body(q_ref, wqk_ref, wv_ref, rot_ref, widx_ref, qkv_ref, keys_ref):
    q = q_ref[...]
    qk = jnp.dot(q, wqk_ref[...], preferred_element_type=jnp.float32)
    v = jnp.dot(q, wv_ref[...], preferred_element_type=jnp.float32)
    row0 = pl.program_id(0) * _RB
    pos = row0 + lax.broadcasted_iota(jnp.int32, (_RB, 1), 0)
    for h in range(H):
        qk_h = qk[:, DH * h:DH * (h + 1)]
        qkv_ref[:, 128 * h:128 * h + 64] = qk_h
        qkv_ref[:, 128 * h + 64:128 * h + 128] = v[:, DH * h:DH * (h + 1)]
        r = jnp.dot(qk_h, rot_ref[...], preferred_element_type=jnp.float32)
        oh = jnp.concatenate(
            [(r[:, 128 * nh:128 * (nh + 1)] >=
              jnp.max(r[:, 128 * nh:128 * (nh + 1)], axis=1, keepdims=True)
              ).astype(jnp.float32) for nh in range(NH)], axis=1)
        idx4 = jnp.dot(oh, widx_ref[...], preferred_element_type=jnp.float32)
        keys_ref[:, NH * h:NH * (h + 1)] = idx4.astype(jnp.int32) * N + pos
    del v


def _proj_hash(queries2d, W_qk, W_v, rot_cat, widx):
    grid = (N // _RB,)
    return pl.pallas_call(
        _proj_hash_body,
        grid=grid,
        in_specs=[
            pl.BlockSpec((_RB, D), lambda i: (i, 0)),
            pl.BlockSpec((D, D), lambda i: (0, 0)),
            pl.BlockSpec((D, D), lambda i: (0, 0)),
            pl.BlockSpec((DH, NH * 128), lambda i: (0, 0)),
            pl.BlockSpec((NH * 128, NH), lambda i: (0, 0)),
        ],
        out_specs=[
            pl.BlockSpec((_RB, H * 128), lambda i: (i, 0)),
            pl.BlockSpec((_RB, H * NH), lambda i: (i, 0)),
        ],
        out_shape=[
            jax.ShapeDtypeStruct((N, H * 128), jnp.float32),
            jax.ShapeDtypeStruct((N, H * NH), jnp.int32),
        ],
    )(queries2d, W_qk, W_v, rot_cat, widx)


def _sc_gather(table, idx, d_row, chunk):
    btot = idx.shape[0]
    nw = 32
    bpw = btot // nw
    steps = bpw // chunk
    mesh = plsc.VectorSubcoreMesh(core_axis_name="c", subcore_axis_name="s")

    @functools.partial(
        pl.kernel, mesh=mesh,
        out_type=jax.ShapeDtypeStruct((btot, d_row), jnp.float32),
        scratch_types=[
            pltpu.VMEM((chunk,), jnp.int32),
            pltpu.VMEM((chunk, d_row), jnp.float32),
            pltpu.SemaphoreType.DMA,
        ],
    )
    def k(table_hbm, idx_hbm, out_hbm, idx_v, rows_v, sem):
        wid = lax.axis_index("s") * 2 + lax.axis_index("c")
        base = wid * bpw

        def body(i, carry):
            off = base + i * chunk
            pltpu.sync_copy(idx_hbm.at[pl.ds(off, chunk)], idx_v)
            pltpu.async_copy(table_hbm.at[idx_v], rows_v, sem).wait()
            pltpu.sync_copy(rows_v, out_hbm.at[pl.ds(off, chunk)])
            return carry

        lax.fori_loop(0, steps, body, 0)

    return k(table, idx)


def _sc_scatter(src, idx, d_row, chunk):
    btot = idx.shape[0]
    nw = 32
    bpw = btot // nw
    steps = bpw // chunk
    mesh = plsc.VectorSubcoreMesh(core_axis_name="c", subcore_axis_name="s")

    @functools.partial(
        pl.kernel, mesh=mesh,
        out_type=jax.ShapeDtypeStruct((btot, d_row), jnp.float32),
        scratch_types=[
            pltpu.VMEM((chunk,), jnp.int32),
            pltpu.VMEM((chunk, d_row), jnp.float32),
            pltpu.SemaphoreType.DMA,
        ],
    )
    def k(src_hbm, idx_hbm, out_hbm, idx_v, rows_v, sem):
        wid = lax.axis_index("s") * 2 + lax.axis_index("c")
        base = wid * bpw

        def body(i, carry):
            off = base + i * chunk
            pltpu.sync_copy(idx_hbm.at[pl.ds(off, chunk)], idx_v)
            pltpu.sync_copy(src_hbm.at[pl.ds(off, chunk)], rows_v)
            pltpu.async_copy(rows_v, out_hbm.at[idx_v], sem).wait()
            return carry

        lax.fori_loop(0, steps, body, 0)

    return k(src, idx)


def _attn_body(sqkv_c, sqkv_p, strow_c, strow_p, stcol_c, so_ref):
    rows = _CPB * BUCKET
    ext_n = rows + BUCKET
    cur = sqkv_c[0]
    prevb = sqkv_p[0]
    trow_c = strow_c[0, 0]
    trow_p = strow_p[0, 0]
    tcol = stcol_c[0, 0]
    scale = DH ** -0.5

    ext = jnp.concatenate([prevb, cur], axis=0)
    ext_k = ext[:, 0:64]
    ext_v = ext[:, 64:128]
    nrm = jnp.sqrt(jnp.sum(ext_k * ext_k, axis=1, keepdims=True))
    ext_k = ext_k / jnp.maximum(nrm, 1e-6)
    bq = cur[:, 0:64]

    dots = lax.dot_general(bq, ext_k, (((1,), (1,)), ((), ())),
                           preferred_element_type=jnp.float32) * scale
    ext_t = jnp.concatenate([trow_p[:, rows - 64:rows], trow_c], axis=1)
    dots = dots - 1e5 * (tcol == ext_t).astype(jnp.float32)
    rb = lax.broadcasted_iota(jnp.int32, (rows, ext_n), 0) // BUCKET
    cb = lax.broadcasted_iota(jnp.int32, (rows, ext_n), 1) // BUCKET
    valid = (cb == rb) | (cb == rb + 1)
    dots = jnp.where(valid, dots, -1e9)
    m = jnp.max(dots, axis=1, keepdims=True)
    p = jnp.exp(dots - m)
    s = jnp.sum(p, axis=1, keepdims=True)
    lse = m + jnp.log(s)
    bo = lax.dot_general(p / s, ext_v, (((1,), (0,)), ((), ())),
                         preferred_element_type=jnp.float32)
    so_ref[0, :, 0:64] = bo
    so_ref[0, :, 64:65] = lse
    so_ref[0, :, 65:_SO_D] = jnp.zeros((rows, _SO_D - 65), jnp.float32)


def _attention(sqkv, st_row4, st_col4):
    nblk = NCH // _CPB
    rows = _CPB * BUCKET
    return pl.pallas_call(
        _attn_body,
        grid=(H, nblk),
        in_specs=[
            pl.BlockSpec((1, rows, 128), lambda h, i: (h, i, 0)),
            pl.BlockSpec((1, 64, 128), lambda h, i: (h, (_CPB * i - 1) % NCH, 0)),
            pl.BlockSpec((1, 1, 1, rows), lambda h, i: (h, i, 0, 0)),
            pl.BlockSpec((1, 1, 1, rows), lambda h, i: (h, (i - 1) % nblk, 0, 0)),
            pl.BlockSpec((1, 1, rows, 1), lambda h, i: (h, i, 0, 0)),
        ],
        out_specs=pl.BlockSpec((1, rows, _SO_D), lambda h, i: (h, i, 0)),
        out_shape=jax.ShapeDtypeStruct((H, TOT, _SO_D), jnp.float32),
    )(sqkv, sqkv, st_row4, st_row4, st_col4)


def _combine_body(o_ref, w_ref, b_ref, out_ref):
    h = pl.program_id(1)
    blk = o_ref[0]
    logit = [blk[i, :, 64:65] for i in range(NH)]
    m = jnp.maximum(jnp.maximum(logit[0], logit[1]),
                    jnp.maximum(logit[2], logit[3]))
    e = [jnp.exp(x - m) for x in logit]
    s = e[0] + e[1] + e[2] + e[3]
    comb = sum(blk[i, :, 0:64] * (e[i] / s) for i in range(NH))
    contrib = jnp.dot(comb, w_ref[...], preferred_element_type=jnp.float32)

    @pl.when(h == 0)
    def _():
        out_ref[...] = contrib + b_ref[...]

    @pl.when(h != 0)
    def _():
        out_ref[...] += contrib


def _combine(o_uns4, W_out, b_out2):
    return pl.pallas_call(
        _combine_body,
        grid=(N // _RB, H),
        in_specs=[
            pl.BlockSpec((1, NH, _RB, _SO_D), lambda i, h: (h, 0, i, 0)),
            pl.BlockSpec((DH, D), lambda i, h: (h, 0)),
            pl.BlockSpec((1, D), lambda i, h: (0, 0)),
        ],
        out_specs=pl.BlockSpec((_RB, D), lambda i, h: (i, 0)),
        out_shape=jax.ShapeDtypeStruct((N, D), jnp.float32),
    )(o_uns4, W_out, b_out2)


def kernel(queries, keys, values, attn_mask, tau, delta, W_qk, W_v, W_out, b_out):
    del keys, values, attn_mask, tau, delta
    q2 = queries.reshape(N, D)

    rot = jax.random.normal(jax.random.key(42), (DH, NH, NB // 2), jnp.float32)
    rot_cat = jnp.concatenate([rot, -rot], axis=-1).reshape(DH, NH * 128)
    lanes = jnp.arange(NH * 128, dtype=jnp.int32)
    widx = jnp.where(lanes[:, None] // 128 == jnp.arange(NH)[None, :],
                     lanes[:, None] % 128, 0).astype(jnp.float32)

    qkv, keys_tok = _proj_hash(q2, W_qk, W_v, rot_cat, widx)

    keys_hm = keys_tok.reshape(N, H, NH).transpose(1, 2, 0)
    skeys = jnp.sort(keys_hm, axis=-1, stable=False)
    st3 = skeys & (N - 1)
    st = st3.reshape(H, TOT)
    sticker = (st3 + (jnp.arange(NH, dtype=jnp.int32) * N)[None, :, None]
               ).reshape(H, TOT)

    gidx1 = (st * H + jnp.arange(H, dtype=jnp.int32)[:, None]).reshape(-1)
    sqkv = _sc_gather(qkv.reshape(N * H, 128), gidx1, 128, 512)
    sqkv = sqkv.reshape(H, TOT, 128)

    st_row4 = st.reshape(H, NCH // _CPB, 1, _CPB * BUCKET)
    st_col4 = st.reshape(H, NCH // _CPB, _CPB * BUCKET, 1)
    so = _attention(sqkv, st_row4, st_col4)

    scidx = (sticker + (jnp.arange(H, dtype=jnp.int32) * TOT)[:, None]).reshape(-1)
    o_uns = _sc_scatter(so.reshape(H * TOT, _SO_D), scidx, _SO_D, 512)
    o_uns4 = o_uns.reshape(H, NH, N, _SO_D)

    out = _combine(o_uns4, W_out, b_out.reshape(1, D))
    return out.reshape(1, N, D)


# speedup vs baseline: 8.1486x; 1.0908x over previous
import functools

import jax
import jax.numpy as jnp
from jax import lax
from jax.experimental import pallas as pl
from jax.experimental.pallas import tpu as pltpu
from jax.experimental.pallas import tpu_sc as plsc

N = 8192
D = 768
H = 12
DH = 64
BUCKET = 64
NH = 4
NB = N // BUCKET
NCH = NH * NB
TOT = NH * N

_RB = 512
_CPB = 8
_SO_D = 128


def _proj_hash_body(q_ref, wqk_ref, wv_ref, rot_ref, widx_ref, qkv_ref, keys_ref):
    q = q_ref[...]
    qk = jnp.dot(q, wqk_ref[...], preferred_element_type=jnp.float32)
    v = jnp.dot(q, wv_ref[...], preferred_element_type=jnp.float32)
    row0 = pl.program_id(0) * _RB
    pos = row0 + lax.broadcasted_iota(jnp.int32, (_RB, 1), 0)
    for h in range(H):
        qk_h = qk[:, DH * h:DH * (h + 1)]
        qkv_ref[:, 128 * h:128 * h + 64] = qk_h
        qkv_ref[:, 128 * h + 64:128 * h + 128] = v[:, DH * h:DH * (h + 1)]
        r = jnp.dot(qk_h, rot_ref[...], preferred_element_type=jnp.float32)
        oh = jnp.concatenate(
            [(r[:, 128 * nh:128 * (nh + 1)] >=
              jnp.max(r[:, 128 * nh:128 * (nh + 1)], axis=1, keepdims=True)
              ).astype(jnp.float32) for nh in range(NH)], axis=1)
        idx4 = jnp.dot(oh, widx_ref[...], preferred_element_type=jnp.float32)
        keys_ref[:, NH * h:NH * (h + 1)] = idx4.astype(jnp.int32) * N + pos
    del v


def _proj_hash(queries2d, W_qk, W_v, rot_cat, widx):
    grid = (N // _RB,)
    return pl.pallas_call(
        _proj_hash_body,
        grid=grid,
        in_specs=[
            pl.BlockSpec((_RB, D), lambda i: (i, 0)),
            pl.BlockSpec((D, D), lambda i: (0, 0)),
            pl.BlockSpec((D, D), lambda i: (0, 0)),
            pl.BlockSpec((DH, NH * 128), lambda i: (0, 0)),
            pl.BlockSpec((NH * 128, NH), lambda i: (0, 0)),
        ],
        out_specs=[
            pl.BlockSpec((_RB, H * 128), lambda i: (i, 0)),
            pl.BlockSpec((_RB, H * NH), lambda i: (i, 0)),
        ],
        out_shape=[
            jax.ShapeDtypeStruct((N, H * 128), jnp.float32),
            jax.ShapeDtypeStruct((N, H * NH), jnp.int32),
        ],
    )(queries2d, W_qk, W_v, rot_cat, widx)


def _sc_gather(table, idx, d_row, chunk):
    btot = idx.shape[0]
    nw = 32
    bpw = btot // nw
    steps = bpw // chunk
    mesh = plsc.VectorSubcoreMesh(core_axis_name="c", subcore_axis_name="s")

    @functools.partial(
        pl.kernel, mesh=mesh,
        out_type=jax.ShapeDtypeStruct((btot, d_row), jnp.float32),
        scratch_types=[
            pltpu.VMEM((chunk,), jnp.int32),
            pltpu.VMEM((chunk, d_row), jnp.float32),
            pltpu.SemaphoreType.DMA,
        ],
    )
    def k(table_hbm, idx_hbm, out_hbm, idx_v, rows_v, sem):
        wid = lax.axis_index("s") * 2 + lax.axis_index("c")
        base = wid * bpw

        def body(i, carry):
            off = base + i * chunk
            pltpu.sync_copy(idx_hbm.at[pl.ds(off, chunk)], idx_v)
            pltpu.async_copy(table_hbm.at[idx_v], rows_v, sem).wait()
            pltpu.sync_copy(rows_v, out_hbm.at[pl.ds(off, chunk)])
            return carry

        lax.fori_loop(0, steps, body, 0)

    return k(table, idx)


def _sc_scatter(src, idx, d_row, chunk):
    btot = idx.shape[0]
    nw = 32
    bpw = btot // nw
    steps = bpw // chunk
    mesh = plsc.VectorSubcoreMesh(core_axis_name="c", subcore_axis_name="s")

    @functools.partial(
        pl.kernel, mesh=mesh,
        out_type=jax.ShapeDtypeStruct((btot, d_row), jnp.float32),
        scratch_types=[
            pltpu.VMEM((chunk,), jnp.int32),
            pltpu.VMEM((chunk, d_row), jnp.float32),
            pltpu.SemaphoreType.DMA,
        ],
    )
    def k(src_hbm, idx_hbm, out_hbm, idx_v, rows_v, sem):
        wid = lax.axis_index("s") * 2 + lax.axis_index("c")
        base = wid * bpw

        def body(i, carry):
            off = base + i * chunk
            pltpu.sync_copy(idx_hbm.at[pl.ds(off, chunk)], idx_v)
            pltpu.sync_copy(src_hbm.at[pl.ds(off, chunk)], rows_v)
            pltpu.async_copy(rows_v, out_hbm.at[idx_v], sem).wait()
            return carry

        lax.fori_loop(0, steps, body, 0)

    return k(src, idx)


def _attn_body(sqkv_c, sqkv_p, strow_c, strow_p, stcol_c, band_ref, so_ref):
    rows = _CPB * BUCKET
    cur = sqkv_c[0]
    prevb = sqkv_p[0]
    trow_c = strow_c[0, 0]
    trow_p = strow_p[0, 0]
    tcol = stcol_c[0, 0]
    band = band_ref[...]
    scale = DH ** -0.5

    ext = jnp.concatenate([prevb, cur], axis=0)
    ext_k = ext[:, 0:64]
    nrm = jnp.sqrt(jnp.sum(ext_k * ext_k, axis=1, keepdims=True))
    ext_k = ext_k / jnp.maximum(nrm, 1e-6)
    ext_t = jnp.concatenate([trow_p[:, rows - 64:rows], trow_c], axis=1)

    for u in range(_CPB // 2):
        r0 = 128 * u
        q_u = cur[r0:r0 + 128, 0:64]
        k_u = ext_k[r0:r0 + 192, :]
        v_u = ext[r0:r0 + 192, 64:128]
        dots = lax.dot_general(q_u, k_u, (((1,), (1,)), ((), ())),
                               preferred_element_type=jnp.float32) * scale
        t_q = tcol[r0:r0 + 128, :]
        t_k = ext_t[:, r0:r0 + 192]
        dots = jnp.where(t_q == t_k, dots - 1e5, dots) + band
        m = jnp.max(dots, axis=1, keepdims=True)
        p = jnp.exp(dots - m)
        s = jnp.sum(p, axis=1, keepdims=True)
        bo = lax.dot_general(p, v_u, (((1,), (0,)), ((), ())),
                             preferred_element_type=jnp.float32) * (1.0 / s)
        so_ref[0, r0:r0 + 128, 0:64] = bo
        so_ref[0, r0:r0 + 128, 64:65] = m + jnp.log(s)


def _attention(sqkv, st_row4, st_col4, band):
    nblk = NCH // _CPB
    rows = _CPB * BUCKET
    return pl.pallas_call(
        _attn_body,
        grid=(H, nblk),
        in_specs=[
            pl.BlockSpec((1, rows, 128), lambda h, i: (h, i, 0)),
            pl.BlockSpec((1, 64, 128), lambda h, i: (h, (_CPB * i - 1) % NCH, 0)),
            pl.BlockSpec((1, 1, 1, rows), lambda h, i: (h, i, 0, 0)),
            pl.BlockSpec((1, 1, 1, rows), lambda h, i: (h, (i - 1) % nblk, 0, 0)),
            pl.BlockSpec((1, 1, rows, 1), lambda h, i: (h, i, 0, 0)),
            pl.BlockSpec((128, 192), lambda h, i: (0, 0)),
        ],
        out_specs=pl.BlockSpec((1, rows, _SO_D), lambda h, i: (h, i, 0)),
        out_shape=jax.ShapeDtypeStruct((H, TOT, _SO_D), jnp.float32),
    )(sqkv, sqkv, st_row4, st_row4, st_col4, band)


def _combine_body(o_ref, w_ref, b_ref, out_ref):
    h = pl.program_id(1)
    blk = o_ref[0]
    logit = [blk[i, :, 64:65] for i in range(NH)]
    m = jnp.maximum(jnp.maximum(logit[0], logit[1]),
                    jnp.maximum(logit[2], logit[3]))
    e = [jnp.exp(x - m) for x in logit]
    s = e[0] + e[1] + e[2] + e[3]
    comb = sum(blk[i, :, 0:64] * (e[i] / s) for i in range(NH))
    contrib = jnp.dot(comb, w_ref[...], preferred_element_type=jnp.float32)

    @pl.when(h == 0)
    def _():
        out_ref[...] = contrib + b_ref[...]

    @pl.when(h != 0)
    def _():
        out_ref[...] += contrib


def _combine(o_uns4, W_out, b_out2):
    return pl.pallas_call(
        _combine_body,
        grid=(N // _RB, H),
        in_specs=[
            pl.BlockSpec((1, NH, _RB, _SO_D), lambda i, h: (h, 0, i, 0)),
            pl.BlockSpec((DH, D), lambda i, h: (h, 0)),
            pl.BlockSpec((1, D), lambda i, h: (0, 0)),
        ],
        out_specs=pl.BlockSpec((_RB, D), lambda i, h: (i, 0)),
        out_shape=jax.ShapeDtypeStruct((N, D), jnp.float32),
    )(o_uns4, W_out, b_out2)


def kernel(queries, keys, values, attn_mask, tau, delta, W_qk, W_v, W_out, b_out):
    del keys, values, attn_mask, tau, delta
    q2 = queries.reshape(N, D)

    rot = jax.random.normal(jax.random.key(42), (DH, NH, NB // 2), jnp.float32)
    rot_cat = jnp.concatenate([rot, -rot], axis=-1).reshape(DH, NH * 128)
    lanes = jnp.arange(NH * 128, dtype=jnp.int32)
    widx = jnp.where(lanes[:, None] // 128 == jnp.arange(NH)[None, :],
                     lanes[:, None] % 128, 0).astype(jnp.float32)

    qkv, keys_tok = _proj_hash(q2, W_qk, W_v, rot_cat, widx)

    keys_hm = keys_tok.reshape(N, H, NH).transpose(1, 2, 0)
    skeys = jnp.sort(keys_hm, axis=-1, stable=False)
    st3 = skeys & (N - 1)
    st = st3.reshape(H, TOT)
    sticker = (st3 + (jnp.arange(NH, dtype=jnp.int32) * N)[None, :, None]
               ).reshape(H, TOT)

    gidx1 = (st * H + jnp.arange(H, dtype=jnp.int32)[:, None]).reshape(-1)
    sqkv = _sc_gather(qkv.reshape(N * H, 128), gidx1, 128, 512)
    sqkv = sqkv.reshape(H, TOT, 128)

    st_row4 = st.reshape(H, NCH // _CPB, 1, _CPB * BUCKET)
    st_col4 = st.reshape(H, NCH // _CPB, _CPB * BUCKET, 1)
    rb = jnp.arange(128, dtype=jnp.int32)[:, None] // BUCKET
    cb = jnp.arange(192, dtype=jnp.int32)[None, :] // BUCKET
    band = jnp.where((cb == rb) | (cb == rb + 1), 0.0, -1e9).astype(jnp.float32)
    so = _attention(sqkv, st_row4, st_col4, band)

    scidx = (sticker + (jnp.arange(H, dtype=jnp.int32) * TOT)[:, None]).reshape(-1)
    o_uns = _sc_scatter(so.reshape(H * TOT, _SO_D), scidx, _SO_D, 512)
    o_uns4 = o_uns.reshape(H, NH, N, _SO_D)

    out = _combine(o_uns4, W_out, b_out.reshape(1, D))
    return out.reshape(1, N, D)


# speedup vs baseline: 8.4091x; 1.0320x over previous
import functools

import jax
import jax.numpy as jnp
from jax import lax
from jax.experimental import pallas as pl
from jax.experimental.pallas import tpu as pltpu
from jax.experimental.pallas import tpu_sc as plsc

N = 8192
D = 768
H = 12
DH = 64
BUCKET = 64
NH = 4
NB = N // BUCKET
NCH = NH * NB
TOT = NH * N

_RB = 512
_CPB = 16
_SO_D = 128


def _proj_hash_body(q_ref, wqk_ref, wv_ref, rot_ref, widx_ref, qkv_ref, keys_ref):
    q = q_ref[...]
    qk = jnp.dot(q, wqk_ref[...], preferred_element_type=jnp.float32)
    v = jnp.dot(q, wv_ref[...], preferred_element_type=jnp.float32)
    row0 = pl.program_id(0) * _RB
    pos = row0 + lax.broadcasted_iota(jnp.int32, (_RB, 1), 0)
    for h in range(H):
        qk_h = qk[:, DH * h:DH * (h + 1)]
        qkv_ref[:, 128 * h:128 * h + 64] = qk_h
        qkv_ref[:, 128 * h + 64:128 * h + 128] = v[:, DH * h:DH * (h + 1)]
        r = jnp.dot(qk_h, rot_ref[...], preferred_element_type=jnp.float32)
        oh = jnp.concatenate(
            [(r[:, 128 * nh:128 * (nh + 1)] >=
              jnp.max(r[:, 128 * nh:128 * (nh + 1)], axis=1, keepdims=True)
              ).astype(jnp.float32) for nh in range(NH)], axis=1)
        idx4 = jnp.dot(oh, widx_ref[...], preferred_element_type=jnp.float32)
        keys_ref[:, NH * h:NH * (h + 1)] = idx4.astype(jnp.int32) * N + pos
    del v


def _proj_hash(queries2d, W_qk, W_v, rot_cat, widx):
    grid = (N // _RB,)
    return pl.pallas_call(
        _proj_hash_body,
        grid=grid,
        in_specs=[
            pl.BlockSpec((_RB, D), lambda i: (i, 0)),
            pl.BlockSpec((D, D), lambda i: (0, 0)),
            pl.BlockSpec((D, D), lambda i: (0, 0)),
            pl.BlockSpec((DH, NH * 128), lambda i: (0, 0)),
            pl.BlockSpec((NH * 128, NH), lambda i: (0, 0)),
        ],
        out_specs=[
            pl.BlockSpec((_RB, H * 128), lambda i: (i, 0)),
            pl.BlockSpec((_RB, H * NH), lambda i: (i, 0)),
        ],
        out_shape=[
            jax.ShapeDtypeStruct((N, H * 128), jnp.float32),
            jax.ShapeDtypeStruct((N, H * NH), jnp.int32),
        ],
    )(queries2d, W_qk, W_v, rot_cat, widx)


def _sc_gather(table, idx, d_row, chunk):
    btot = idx.shape[0]
    nw = 32
    bpw = btot // nw
    steps = bpw // chunk
    mesh = plsc.VectorSubcoreMesh(core_axis_name="c", subcore_axis_name="s")

    @functools.partial(
        pl.kernel, mesh=mesh,
        out_type=jax.ShapeDtypeStruct((btot, d_row), jnp.float32),
        scratch_types=[
            pltpu.VMEM((chunk,), jnp.int32),
            pltpu.VMEM((chunk, d_row), jnp.float32),
            pltpu.SemaphoreType.DMA,
        ],
    )
    def k(table_hbm, idx_hbm, out_hbm, idx_v, rows_v, sem):
        wid = lax.axis_index("s") * 2 + lax.axis_index("c")
        base = wid * bpw

        def body(i, carry):
            off = base + i * chunk
            pltpu.sync_copy(idx_hbm.at[pl.ds(off, chunk)], idx_v)
            pltpu.async_copy(table_hbm.at[idx_v], rows_v, sem).wait()
            pltpu.sync_copy(rows_v, out_hbm.at[pl.ds(off, chunk)])
            return carry

        lax.fori_loop(0, steps, body, 0)

    return k(table, idx)


def _sc_scatter(src, idx, d_row, chunk):
    btot = idx.shape[0]
    nw = 32
    bpw = btot // nw
    steps = bpw // chunk
    mesh = plsc.VectorSubcoreMesh(core_axis_name="c", subcore_axis_name="s")

    @functools.partial(
        pl.kernel, mesh=mesh,
        out_type=jax.ShapeDtypeStruct((btot, d_row), jnp.float32),
        scratch_types=[
            pltpu.VMEM((chunk,), jnp.int32),
            pltpu.VMEM((chunk, d_row), jnp.float32),
            pltpu.SemaphoreType.DMA,
        ],
    )
    def k(src_hbm, idx_hbm, out_hbm, idx_v, rows_v, sem):
        wid = lax.axis_index("s") * 2 + lax.axis_index("c")
        base = wid * bpw

        def body(i, carry):
            off = base + i * chunk
            pltpu.sync_copy(idx_hbm.at[pl.ds(off, chunk)], idx_v)
            pltpu.sync_copy(src_hbm.at[pl.ds(off, chunk)], rows_v)
            pltpu.async_copy(rows_v, out_hbm.at[idx_v], sem).wait()
            return carry

        lax.fori_loop(0, steps, body, 0)

    return k(src, idx)


def _attn_body(sqkv_c, sqkv_p, strow_c, strow_p, stcol_c, band_ref, so_ref):
    rows = _CPB * BUCKET
    cur = sqkv_c[0]
    prevb = sqkv_p[0]
    trow_c = strow_c[0, 0]
    trow_p = strow_p[0, 0]
    tcol = stcol_c[0, 0]
    band = band_ref[...]
    scale = DH ** -0.5

    ext = jnp.concatenate([prevb, cur], axis=0)
    ext_k = ext[:, 0:64]
    nrm = jnp.sqrt(jnp.sum(ext_k * ext_k, axis=1, keepdims=True))
    ext_k = ext_k / jnp.maximum(nrm, 1e-6)
    ext_t = jnp.concatenate([trow_p[:, rows - 64:rows], trow_c], axis=1)

    for u in range(_CPB // 2):
        r0 = 128 * u
        q_u = cur[r0:r0 + 128, 0:64]
        k_u = ext_k[r0:r0 + 192, :]
        v_u = ext[r0:r0 + 192, 64:128]
        dots = lax.dot_general(q_u, k_u, (((1,), (1,)), ((), ())),
                               preferred_element_type=jnp.float32) * scale
        t_q = tcol[r0:r0 + 128, :]
        t_k = ext_t[:, r0:r0 + 192]
        dots = jnp.where(t_q == t_k, dots - 1e5, dots) + band
        m = jnp.max(dots, axis=1, keepdims=True)
        p = jnp.exp(dots - m)
        s = jnp.sum(p, axis=1, keepdims=True)
        bo = lax.dot_general(p, v_u, (((1,), (0,)), ((), ())),
                             preferred_element_type=jnp.float32) * (1.0 / s)
        so_ref[0, r0:r0 + 128, 0:64] = bo
        so_ref[0, r0:r0 + 128, 64:65] = m + jnp.log(s)


def _attention(sqkv, st_row4, st_col4, band):
    nblk = NCH // _CPB
    rows = _CPB * BUCKET
    return pl.pallas_call(
        _attn_body,
        grid=(H, nblk),
        in_specs=[
            pl.BlockSpec((1, rows, 128), lambda h, i: (h, i, 0)),
            pl.BlockSpec((1, 64, 128), lambda h, i: (h, (_CPB * i - 1) % NCH, 0)),
            pl.BlockSpec((1, 1, 1, rows), lambda h, i: (h, i, 0, 0)),
            pl.BlockSpec((1, 1, 1, rows), lambda h, i: (h, (i - 1) % nblk, 0, 0)),
            pl.BlockSpec((1, 1, rows, 1), lambda h, i: (h, i, 0, 0)),
            pl.BlockSpec((128, 192), lambda h, i: (0, 0)),
        ],
        out_specs=pl.BlockSpec((1, rows, _SO_D), lambda h, i: (h, i, 0)),
        out_shape=jax.ShapeDtypeStruct((H, TOT, _SO_D), jnp.float32),
    )(sqkv, sqkv, st_row4, st_row4, st_col4, band)


def _combine_body(o_ref, w_ref, b_ref, out_ref):
    h = pl.program_id(1)
    blk = o_ref[0]
    logit = [blk[i, :, 64:65] for i in range(NH)]
    m = jnp.maximum(jnp.maximum(logit[0], logit[1]),
                    jnp.maximum(logit[2], logit[3]))
    e = [jnp.exp(x - m) for x in logit]
    s = e[0] + e[1] + e[2] + e[3]
    comb = sum(blk[i, :, 0:64] * (e[i] / s) for i in range(NH))
    contrib = jnp.dot(comb, w_ref[...], preferred_element_type=jnp.float32)

    @pl.when(h == 0)
    def _():
        out_ref[...] = contrib + b_ref[...]

    @pl.when(h != 0)
    def _():
        out_ref[...] += contrib


def _combine(o_uns4, W_out, b_out2):
    return pl.pallas_call(
        _combine_body,
        grid=(N // _RB, H),
        in_specs=[
            pl.BlockSpec((1, NH, _RB, _SO_D), lambda i, h: (h, 0, i, 0)),
            pl.BlockSpec((DH, D), lambda i, h: (h, 0)),
            pl.BlockSpec((1, D), lambda i, h: (0, 0)),
        ],
        out_specs=pl.BlockSpec((_RB, D), lambda i, h: (i, 0)),
        out_shape=jax.ShapeDtypeStruct((N, D), jnp.float32),
    )(o_uns4, W_out, b_out2)


def kernel(queries, keys, values, attn_mask, tau, delta, W_qk, W_v, W_out, b_out):
    del keys, values, attn_mask, tau, delta
    q2 = queries.reshape(N, D)

    rot = jax.random.normal(jax.random.key(42), (DH, NH, NB // 2), jnp.float32)
    rot_cat = jnp.concatenate([rot, -rot], axis=-1).reshape(DH, NH * 128)
    lanes = jnp.arange(NH * 128, dtype=jnp.int32)
    widx = jnp.where(lanes[:, None] // 128 == jnp.arange(NH)[None, :],
                     lanes[:, None] % 128, 0).astype(jnp.float32)

    qkv, keys_tok = _proj_hash(q2, W_qk, W_v, rot_cat, widx)

    keys_hm = keys_tok.reshape(N, H, NH).transpose(1, 2, 0)
    skeys = jnp.sort(keys_hm, axis=-1, stable=False)
    st3 = skeys & (N - 1)
    st = st3.reshape(H, TOT)
    sticker = (st3 + (jnp.arange(NH, dtype=jnp.int32) * N)[None, :, None]
               ).reshape(H, TOT)

    gidx1 = (st * H + jnp.arange(H, dtype=jnp.int32)[:, None]).reshape(-1)
    sqkv = _sc_gather(qkv.reshape(N * H, 128), gidx1, 128, 512)
    sqkv = sqkv.reshape(H, TOT, 128)

    st_row4 = st.reshape(H, NCH // _CPB, 1, _CPB * BUCKET)
    st_col4 = st.reshape(H, NCH // _CPB, _CPB * BUCKET, 1)
    rb = jnp.arange(128, dtype=jnp.int32)[:, None] // BUCKET
    cb = jnp.arange(192, dtype=jnp.int32)[None, :] // BUCKET
    band = jnp.where((cb == rb) | (cb == rb + 1), 0.0, -1e9).astype(jnp.float32)
    so = _attention(sqkv, st_row4, st_col4, band)

    scidx = (sticker + (jnp.arange(H, dtype=jnp.int32) * TOT)[:, None]).reshape(-1)
    o_uns = _sc_scatter(so.reshape(H * TOT, _SO_D), scidx, _SO_D, 512)
    o_uns4 = o_uns.reshape(H, NH, N, _SO_D)

    out = _combine(o_uns4, W_out, b_out.reshape(1, D))
    return out.reshape(1, N, D)


# speedup vs baseline: 8.4267x; 1.0021x over previous
import functools

import jax
import jax.numpy as jnp
from jax import lax
from jax.experimental import pallas as pl
from jax.experimental.pallas import tpu as pltpu
from jax.experimental.pallas import tpu_sc as plsc

N = 8192
D = 768
H = 12
DH = 64
BUCKET = 64
NH = 4
NB = N // BUCKET
NCH = NH * NB
TOT = NH * N

_RB = 512
_CPB = 16
_SO_D = 128


def _proj_hash_body(q_ref, wqk_ref, wv_ref, rot_ref, widx_ref, qkv_ref, keys_ref):
    q = q_ref[...]
    qk = jnp.dot(q, wqk_ref[...], preferred_element_type=jnp.float32)
    v = jnp.dot(q, wv_ref[...], preferred_element_type=jnp.float32)
    row0 = pl.program_id(0) * _RB
    pos = row0 + lax.broadcasted_iota(jnp.int32, (_RB, 1), 0)
    for h in range(H):
        qk_h = qk[:, DH * h:DH * (h + 1)]
        qkv_ref[:, 128 * h:128 * h + 64] = qk_h
        qkv_ref[:, 128 * h + 64:128 * h + 128] = v[:, DH * h:DH * (h + 1)]
        r = jnp.dot(qk_h, rot_ref[...], preferred_element_type=jnp.float32)
        oh = jnp.concatenate(
            [(r[:, 128 * nh:128 * (nh + 1)] >=
              jnp.max(r[:, 128 * nh:128 * (nh + 1)], axis=1, keepdims=True)
              ).astype(jnp.float32) for nh in range(NH)], axis=1)
        idx4 = jnp.dot(oh, widx_ref[...], preferred_element_type=jnp.float32)
        keys_ref[:, NH * h:NH * (h + 1)] = idx4.astype(jnp.int32) * N + pos
    del v


def _proj_hash(queries2d, W_qk, W_v, rot_cat, widx):
    grid = (N // _RB,)
    return pl.pallas_call(
        _proj_hash_body,
        grid=grid,
        in_specs=[
            pl.BlockSpec((_RB, D), lambda i: (i, 0)),
            pl.BlockSpec((D, D), lambda i: (0, 0)),
            pl.BlockSpec((D, D), lambda i: (0, 0)),
            pl.BlockSpec((DH, NH * 128), lambda i: (0, 0)),
            pl.BlockSpec((NH * 128, NH), lambda i: (0, 0)),
        ],
        out_specs=[
            pl.BlockSpec((_RB, H * 128), lambda i: (i, 0)),
            pl.BlockSpec((_RB, H * NH), lambda i: (i, 0)),
        ],
        out_shape=[
            jax.ShapeDtypeStruct((N, H * 128), jnp.float32),
            jax.ShapeDtypeStruct((N, H * NH), jnp.int32),
        ],
    )(queries2d, W_qk, W_v, rot_cat, widx)


def _sc_gather(table, idx, d_row, chunk):
    btot = idx.shape[0]
    nw = 32
    bpw = btot // nw
    steps = bpw // chunk
    pairs = steps // 2
    mesh = plsc.VectorSubcoreMesh(core_axis_name="c", subcore_axis_name="s")

    @functools.partial(
        pl.kernel, mesh=mesh,
        out_type=jax.ShapeDtypeStruct((btot, d_row), jnp.float32),
        scratch_types=[
            pltpu.VMEM((chunk,), jnp.int32),
            pltpu.VMEM((chunk,), jnp.int32),
            pltpu.VMEM((chunk, d_row), jnp.float32),
            pltpu.VMEM((chunk, d_row), jnp.float32),
            pltpu.SemaphoreType.DMA,
            pltpu.SemaphoreType.DMA,
        ],
    )
    def k(table_hbm, idx_hbm, out_hbm, idx_v0, idx_v1, rows_v0, rows_v1,
          sg0, sg1):
        wid = lax.axis_index("s") * 2 + lax.axis_index("c")
        base = wid * bpw

        pltpu.sync_copy(idx_hbm.at[pl.ds(base, chunk)], idx_v0)
        pltpu.async_copy(table_hbm.at[idx_v0], rows_v0, sg0)

        def body(ii, carry):
            i0 = 2 * ii
            pltpu.sync_copy(idx_hbm.at[pl.ds(base + (i0 + 1) * chunk, chunk)],
                            idx_v1)
            pltpu.async_copy(table_hbm.at[idx_v1], rows_v1, sg1)
            pltpu.make_async_copy(table_hbm.at[idx_v0], rows_v0, sg0).wait()
            pltpu.sync_copy(rows_v0, out_hbm.at[pl.ds(base + i0 * chunk, chunk)])

            @pl.when(ii + 1 < pairs)
            def _():
                pltpu.sync_copy(
                    idx_hbm.at[pl.ds(base + (i0 + 2) * chunk, chunk)], idx_v0)
                pltpu.async_copy(table_hbm.at[idx_v0], rows_v0, sg0)

            pltpu.make_async_copy(table_hbm.at[idx_v1], rows_v1, sg1).wait()
            pltpu.sync_copy(rows_v1,
                            out_hbm.at[pl.ds(base + (i0 + 1) * chunk, chunk)])
            return carry

        lax.fori_loop(0, pairs, body, 0)

    return k(table, idx)


def _sc_scatter(src, idx, d_row, chunk):
    btot = idx.shape[0]
    nw = 32
    bpw = btot // nw
    steps = bpw // chunk
    mesh = plsc.VectorSubcoreMesh(core_axis_name="c", subcore_axis_name="s")

    @functools.partial(
        pl.kernel, mesh=mesh,
        out_type=jax.ShapeDtypeStruct((btot, d_row), jnp.float32),
        scratch_types=[
            pltpu.VMEM((chunk,), jnp.int32),
            pltpu.VMEM((chunk, d_row), jnp.float32),
            pltpu.SemaphoreType.DMA,
        ],
    )
    def k(src_hbm, idx_hbm, out_hbm, idx_v, rows_v, sem):
        wid = lax.axis_index("s") * 2 + lax.axis_index("c")
        base = wid * bpw

        def body(i, carry):
            off = base + i * chunk
            pltpu.sync_copy(idx_hbm.at[pl.ds(off, chunk)], idx_v)
            pltpu.sync_copy(src_hbm.at[pl.ds(off, chunk)], rows_v)
            pltpu.async_copy(rows_v, out_hbm.at[idx_v], sem).wait()
            return carry

        lax.fori_loop(0, steps, body, 0)

    return k(src, idx)


def _attn_body(sqkv_c, sqkv_p, strow_c, strow_p, stcol_c, band_ref, so_ref):
    rows = _CPB * BUCKET
    cur = sqkv_c[0]
    prevb = sqkv_p[0]
    trow_c = strow_c[0, 0]
    trow_p = strow_p[0, 0]
    tcol = stcol_c[0, 0]
    band = band_ref[...]
    scale = DH ** -0.5

    ext = jnp.concatenate([prevb, cur], axis=0)
    ext_k = ext[:, 0:64]
    nrm = jnp.sqrt(jnp.sum(ext_k * ext_k, axis=1, keepdims=True))
    ext_k = ext_k / jnp.maximum(nrm, 1e-6)
    ext_t = jnp.concatenate([trow_p[:, rows - 64:rows], trow_c], axis=1)

    for u in range(_CPB // 2):
        r0 = 128 * u
        q_u = cur[r0:r0 + 128, 0:64]
        k_u = ext_k[r0:r0 + 192, :]
        v_u = ext[r0:r0 + 192, 64:128]
        dots = lax.dot_general(q_u, k_u, (((1,), (1,)), ((), ())),
                               preferred_element_type=jnp.float32) * scale
        t_q = tcol[r0:r0 + 128, :]
        t_k = ext_t[:, r0:r0 + 192]
        dots = jnp.where(t_q == t_k, dots - 1e5, dots) + band
        m = jnp.max(dots, axis=1, keepdims=True)
        p = jnp.exp(dots - m)
        s = jnp.sum(p, axis=1, keepdims=True)
        bo = lax.dot_general(p, v_u, (((1,), (0,)), ((), ())),
                             preferred_element_type=jnp.float32) / s
        so_ref[0, r0:r0 + 128, 0:64] = bo
        so_ref[0, r0:r0 + 128, 64:65] = m + jnp.log(s)


def _attention(sqkv, st_row4, st_col4, band):
    nblk = NCH // _CPB
    rows = _CPB * BUCKET
    return pl.pallas_call(
        _attn_body,
        grid=(H, nblk),
        in_specs=[
            pl.BlockSpec((1, rows, 128), lambda h, i: (h, i, 0)),
            pl.BlockSpec((1, 64, 128), lambda h, i: (h, (_CPB * i - 1) % NCH, 0)),
            pl.BlockSpec((1, 1, 1, rows), lambda h, i: (h, i, 0, 0)),
            pl.BlockSpec((1, 1, 1, rows), lambda h, i: (h, (i - 1) % nblk, 0, 0)),
            pl.BlockSpec((1, 1, rows, 1), lambda h, i: (h, i, 0, 0)),
            pl.BlockSpec((128, 192), lambda h, i: (0, 0)),
        ],
        out_specs=pl.BlockSpec((1, rows, _SO_D), lambda h, i: (h, i, 0)),
        out_shape=jax.ShapeDtypeStruct((H, TOT, _SO_D), jnp.float32),
    )(sqkv, sqkv, st_row4, st_row4, st_col4, band)


def _combine_body(o_ref, w_ref, b_ref, out_ref):
    h = pl.program_id(1)
    blk = o_ref[0]
    logit = [blk[i, :, 64:65] for i in range(NH)]
    m = jnp.maximum(jnp.maximum(logit[0], logit[1]),
                    jnp.maximum(logit[2], logit[3]))
    e = [jnp.exp(x - m) for x in logit]
    s = e[0] + e[1] + e[2] + e[3]
    comb = sum(blk[i, :, 0:64] * (e[i] / s) for i in range(NH))
    contrib = jnp.dot(comb, w_ref[...], preferred_element_type=jnp.float32)

    @pl.when(h == 0)
    def _():
        out_ref[...] = contrib + b_ref[...]

    @pl.when(h != 0)
    def _():
        out_ref[...] += contrib


def _combine(o_uns4, W_out, b_out2):
    return pl.pallas_call(
        _combine_body,
        grid=(N // _RB, H),
        in_specs=[
            pl.BlockSpec((1, NH, _RB, _SO_D), lambda i, h: (h, 0, i, 0)),
            pl.BlockSpec((DH, D), lambda i, h: (h, 0)),
            pl.BlockSpec((1, D), lambda i, h: (0, 0)),
        ],
        out_specs=pl.BlockSpec((_RB, D), lambda i, h: (i, 0)),
        out_shape=jax.ShapeDtypeStruct((N, D), jnp.float32),
    )(o_uns4, W_out, b_out2)


def kernel(queries, keys, values, attn_mask, tau, delta, W_qk, W_v, W_out, b_out):
    del keys, values, attn_mask, tau, delta
    q2 = queries.reshape(N, D)

    rot = jax.random.normal(jax.random.key(42), (DH, NH, NB // 2), jnp.float32)
    rot_cat = jnp.concatenate([rot, -rot], axis=-1).reshape(DH, NH * 128)
    lanes = jnp.arange(NH * 128, dtype=jnp.int32)
    widx = jnp.where(lanes[:, None] // 128 == jnp.arange(NH)[None, :],
                     lanes[:, None] % 128, 0).astype(jnp.float32)

    qkv, keys_tok = _proj_hash(q2, W_qk, W_v, rot_cat, widx)

    keys_hm = keys_tok.reshape(N, H, NH).transpose(1, 2, 0)
    skeys = jnp.sort(keys_hm, axis=-1, stable=False)
    st3 = skeys & (N - 1)
    st = st3.reshape(H, TOT)
    sticker = (st3 + (jnp.arange(NH, dtype=jnp.int32) * N)[None, :, None]
               ).reshape(H, TOT)

    gidx1 = (st * H + jnp.arange(H, dtype=jnp.int32)[:, None]).reshape(-1)
    sqkv = _sc_gather(qkv.reshape(N * H, 128), gidx1, 128, 384)
    sqkv = sqkv.reshape(H, TOT, 128)

    st_row4 = st.reshape(H, NCH // _CPB, 1, _CPB * BUCKET)
    st_col4 = st.reshape(H, NCH // _CPB, _CPB * BUCKET, 1)
    rb = jnp.arange(128, dtype=jnp.int32)[:, None] // BUCKET
    cb = jnp.arange(192, dtype=jnp.int32)[None, :] // BUCKET
    band = jnp.where((cb == rb) | (cb == rb + 1), 0.0, -1e9).astype(jnp.float32)
    so = _attention(sqkv, st_row4, st_col4, band)

    scidx = (sticker + (jnp.arange(H, dtype=jnp.int32) * TOT)[:, None]).reshape(-1)
    o_uns = _sc_scatter(so.reshape(H * TOT, _SO_D), scidx, _SO_D, 512)
    o_uns4 = o_uns.reshape(H, NH, N, _SO_D)

    out = _combine(o_uns4, W_out, b_out.reshape(1, D))
    return out.reshape(1, N, D)


# speedup vs baseline: 8.4557x; 1.0034x over previous
import functools

import jax
import jax.numpy as jnp
from jax import lax
from jax.experimental import pallas as pl
from jax.experimental.pallas import tpu as pltpu
from jax.experimental.pallas import tpu_sc as plsc

N = 8192
D = 768
H = 12
DH = 64
BUCKET = 64
NH = 4
NB = N // BUCKET
NCH = NH * NB
TOT = NH * N

_RB = 512
_CPB = 16
_SO_D = 128


def _proj_hash_body(q_ref, wqk_ref, wv_ref, rot_ref, widx_ref, qkv_ref, keys_ref):
    q = q_ref[...]
    qk = jnp.dot(q, wqk_ref[...], preferred_element_type=jnp.float32)
    v = jnp.dot(q, wv_ref[...], preferred_element_type=jnp.float32)
    row0 = pl.program_id(0) * _RB
    pos = row0 + lax.broadcasted_iota(jnp.int32, (_RB, 1), 0)
    for h in range(H):
        qk_h = qk[:, DH * h:DH * (h + 1)]
        qkv_ref[:, 128 * h:128 * h + 64] = qk_h
        qkv_ref[:, 128 * h + 64:128 * h + 128] = v[:, DH * h:DH * (h + 1)]
        r = jnp.dot(qk_h, rot_ref[...], preferred_element_type=jnp.float32)
        oh = jnp.concatenate(
            [(r[:, 128 * nh:128 * (nh + 1)] >=
              jnp.max(r[:, 128 * nh:128 * (nh + 1)], axis=1, keepdims=True)
              ).astype(jnp.float32) for nh in range(NH)], axis=1)
        idx4 = jnp.dot(oh, widx_ref[...], preferred_element_type=jnp.float32)
        keys_ref[:, NH * h:NH * (h + 1)] = idx4.astype(jnp.int32) * N + pos
    del v


def _proj_hash(queries2d, W_qk, W_v, rot_cat, widx):
    grid = (N // _RB,)
    return pl.pallas_call(
        _proj_hash_body,
        grid=grid,
        in_specs=[
            pl.BlockSpec((_RB, D), lambda i: (i, 0)),
            pl.BlockSpec((D, D), lambda i: (0, 0)),
            pl.BlockSpec((D, D), lambda i: (0, 0)),
            pl.BlockSpec((DH, NH * 128), lambda i: (0, 0)),
            pl.BlockSpec((NH * 128, NH), lambda i: (0, 0)),
        ],
        out_specs=[
            pl.BlockSpec((_RB, H * 128), lambda i: (i, 0)),
            pl.BlockSpec((_RB, H * NH), lambda i: (i, 0)),
        ],
        out_shape=[
            jax.ShapeDtypeStruct((N, H * 128), jnp.float32),
            jax.ShapeDtypeStruct((N, H * NH), jnp.int32),
        ],
    )(queries2d, W_qk, W_v, rot_cat, widx)


def _sc_gather(table, idx, d_row, chunk):
    btot = idx.shape[0]
    nw = 32
    bpw = btot // nw
    steps = bpw // chunk
    pairs = steps // 2
    mesh = plsc.VectorSubcoreMesh(core_axis_name="c", subcore_axis_name="s")

    @functools.partial(
        pl.kernel, mesh=mesh,
        out_type=jax.ShapeDtypeStruct((btot, d_row), jnp.float32),
        scratch_types=[
            pltpu.VMEM((chunk,), jnp.int32),
            pltpu.VMEM((chunk,), jnp.int32),
            pltpu.VMEM((chunk, d_row), jnp.float32),
            pltpu.VMEM((chunk, d_row), jnp.float32),
            pltpu.SemaphoreType.DMA,
            pltpu.SemaphoreType.DMA,
        ],
    )
    def k(table_hbm, idx_hbm, out_hbm, idx_v0, idx_v1, rows_v0, rows_v1,
          sg0, sg1):
        wid = lax.axis_index("s") * 2 + lax.axis_index("c")
        base = wid * bpw

        pltpu.sync_copy(idx_hbm.at[pl.ds(base, chunk)], idx_v0)
        pltpu.async_copy(table_hbm.at[idx_v0], rows_v0, sg0)

        def body(ii, carry):
            i0 = 2 * ii
            pltpu.sync_copy(idx_hbm.at[pl.ds(base + (i0 + 1) * chunk, chunk)],
                            idx_v1)
            pltpu.async_copy(table_hbm.at[idx_v1], rows_v1, sg1)
            pltpu.make_async_copy(table_hbm.at[idx_v0], rows_v0, sg0).wait()
            pltpu.sync_copy(rows_v0, out_hbm.at[pl.ds(base + i0 * chunk, chunk)])

            @pl.when(ii + 1 < pairs)
            def _():
                pltpu.sync_copy(
                    idx_hbm.at[pl.ds(base + (i0 + 2) * chunk, chunk)], idx_v0)
                pltpu.async_copy(table_hbm.at[idx_v0], rows_v0, sg0)

            pltpu.make_async_copy(table_hbm.at[idx_v1], rows_v1, sg1).wait()
            pltpu.sync_copy(rows_v1,
                            out_hbm.at[pl.ds(base + (i0 + 1) * chunk, chunk)])
            return carry

        lax.fori_loop(0, pairs, body, 0)

    return k(table, idx)


def _sc_scatter(src, idx, d_row, chunk):
    btot = idx.shape[0]
    nw = 32
    bpw = btot // nw
    steps = bpw // chunk
    mesh = plsc.VectorSubcoreMesh(core_axis_name="c", subcore_axis_name="s")

    @functools.partial(
        pl.kernel, mesh=mesh,
        out_type=jax.ShapeDtypeStruct((btot, d_row), jnp.float32),
        scratch_types=[
            pltpu.VMEM((chunk,), jnp.int32),
            pltpu.VMEM((chunk, d_row), jnp.float32),
            pltpu.SemaphoreType.DMA,
        ],
    )
    def k(src_hbm, idx_hbm, out_hbm, idx_v, rows_v, sem):
        wid = lax.axis_index("s") * 2 + lax.axis_index("c")
        base = wid * bpw

        def body(i, carry):
            off = base + i * chunk
            pltpu.sync_copy(idx_hbm.at[pl.ds(off, chunk)], idx_v)
            pltpu.sync_copy(src_hbm.at[pl.ds(off, chunk)], rows_v)
            pltpu.async_copy(rows_v, out_hbm.at[idx_v], sem).wait()
            return carry

        lax.fori_loop(0, steps, body, 0)

    return k(src, idx)


def _attn_body(sqkv_c, sqkv_p, strow_c, strow_p, stcol_c, band_ref, so_ref):
    rows = _CPB * BUCKET
    cur = sqkv_c[0]
    prevb = sqkv_p[0]
    trow_c = strow_c[0, 0]
    trow_p = strow_p[0, 0]
    tcol = stcol_c[0, 0]
    band = band_ref[...]
    scale = DH ** -0.5

    ext = jnp.concatenate([prevb, cur], axis=0)
    ext_k = ext[:, 0:64]
    nrm = jnp.sqrt(jnp.sum(ext_k * ext_k, axis=1, keepdims=True))
    ext_k = ext_k / jnp.maximum(nrm, 1e-6)
    ext_t = jnp.concatenate([trow_p[:, rows - 64:rows], trow_c], axis=1)

    for u in range(_CPB // 2):
        r0 = 128 * u
        q_u = cur[r0:r0 + 128, 0:64]
        k_u = ext_k[r0:r0 + 192, :]
        v_u = ext[r0:r0 + 192, 64:128]
        dots = lax.dot_general(q_u, k_u, (((1,), (1,)), ((), ())),
                               preferred_element_type=jnp.float32) * scale
        t_q = tcol[r0:r0 + 128, :]
        t_k = ext_t[:, r0:r0 + 192]
        dots = jnp.where(t_q == t_k, dots - 1e5, dots) + band
        m = jnp.max(dots, axis=1, keepdims=True)
        p = jnp.exp(dots - m)
        s = jnp.sum(p, axis=1, keepdims=True)
        bo = lax.dot_general(p, v_u, (((1,), (0,)), ((), ())),
                             preferred_element_type=jnp.float32) / s
        so_ref[0, r0:r0 + 128, 0:64] = bo
        so_ref[0, r0:r0 + 128, 64:65] = m + jnp.log(s)


def _attention(sqkv, st_row4, st_col4, band):
    nblk = NCH // _CPB
    rows = _CPB * BUCKET
    return pl.pallas_call(
        _attn_body,
        grid=(H, nblk),
        in_specs=[
            pl.BlockSpec((1, rows, 128), lambda h, i: (h, i, 0)),
            pl.BlockSpec((1, 64, 128), lambda h, i: (h, (_CPB * i - 1) % NCH, 0)),
            pl.BlockSpec((1, 1, 1, rows), lambda h, i: (h, i, 0, 0)),
            pl.BlockSpec((1, 1, 1, rows), lambda h, i: (h, (i - 1) % nblk, 0, 0)),
            pl.BlockSpec((1, 1, rows, 1), lambda h, i: (h, i, 0, 0)),
            pl.BlockSpec((128, 192), lambda h, i: (0, 0)),
        ],
        out_specs=pl.BlockSpec((1, rows, _SO_D), lambda h, i: (h, i, 0)),
        out_shape=jax.ShapeDtypeStruct((H, TOT, _SO_D), jnp.float32),
    )(sqkv, sqkv, st_row4, st_row4, st_col4, band)


def _combine_body(o_ref, w_ref, b_ref, out_ref):
    h = pl.program_id(1)
    blk = o_ref[0]
    logit = [blk[i, :, 64:65] for i in range(NH)]
    m = jnp.maximum(jnp.maximum(logit[0], logit[1]),
                    jnp.maximum(logit[2], logit[3]))
    e = [jnp.exp(x - m) for x in logit]
    s = e[0] + e[1] + e[2] + e[3]
    comb = sum(blk[i, :, 0:64] * (e[i] / s) for i in range(NH))
    contrib = jnp.dot(comb, w_ref[...], preferred_element_type=jnp.float32)

    @pl.when(h == 0)
    def _():
        out_ref[...] = contrib + b_ref[...]

    @pl.when(h != 0)
    def _():
        out_ref[...] += contrib


def _combine(o_uns4, W_out, b_out2):
    return pl.pallas_call(
        _combine_body,
        grid=(N // _RB, H),
        in_specs=[
            pl.BlockSpec((1, NH, _RB, _SO_D), lambda i, h: (h, 0, i, 0)),
            pl.BlockSpec((DH, D), lambda i, h: (h, 0)),
            pl.BlockSpec((1, D), lambda i, h: (0, 0)),
        ],
        out_specs=pl.BlockSpec((_RB, D), lambda i, h: (i, 0)),
        out_shape=jax.ShapeDtypeStruct((N, D), jnp.float32),
    )(o_uns4, W_out, b_out2)


def kernel(queries, keys, values, attn_mask, tau, delta, W_qk, W_v, W_out, b_out):
    del keys, values, attn_mask, tau, delta
    q2 = queries.reshape(N, D)

    rot = jax.random.normal(jax.random.key(42), (DH, NH, NB // 2), jnp.float32)
    rot_cat = jnp.concatenate([rot, -rot], axis=-1).reshape(DH, NH * 128)
    lanes = jnp.arange(NH * 128, dtype=jnp.int32)
    widx = jnp.where(lanes[:, None] // 128 == jnp.arange(NH)[None, :],
                     lanes[:, None] % 128, 0).astype(jnp.float32)

    qkv, keys_tok = _proj_hash(q2, W_qk, W_v, rot_cat, widx)

    keys_hm = keys_tok.reshape(N, H, NH).transpose(1, 2, 0)
    skeys = jnp.sort(keys_hm, axis=-1, stable=False)
    st3 = skeys & (N - 1)
    st = st3.reshape(H, TOT)
    sticker = (st3 + (jnp.arange(NH, dtype=jnp.int32) * N)[None, :, None]
               ).reshape(H, TOT)

    gidx1 = (st * H + jnp.arange(H, dtype=jnp.int32)[:, None]).reshape(-1)
    sqkv = _sc_gather(qkv.reshape(N * H, 128), gidx1, 128, 384)
    sqkv = sqkv.reshape(H, TOT, 128)

    st_row4 = st.reshape(H, NCH // _CPB, 1, _CPB * BUCKET)
    st_col4 = st.reshape(H, NCH // _CPB, _CPB * BUCKET, 1)
    rb = jnp.arange(128, dtype=jnp.int32)[:, None] // BUCKET
    cb = jnp.arange(192, dtype=jnp.int32)[None, :] // BUCKET
    band = jnp.where((cb == rb) | (cb == rb + 1), 0.0, -1e9).astype(jnp.float32)
    so = _attention(sqkv, st_row4, st_col4, band)

    scidx = (sticker + (jnp.arange(H, dtype=jnp.int32) * TOT)[:, None]).reshape(-1)
    o_uns = _sc_scatter(so.reshape(H * TOT, _SO_D), scidx, _SO_D, 768)
    o_uns4 = o_uns.reshape(H, NH, N, _SO_D)

    out = _combine(o_uns4, W_out, b_out.reshape(1, D))
    return out.reshape(1, N, D)
